# baseline (speedup 1.0000x reference)
import functools

import jax
import jax.numpy as jnp
from jax import lax
from jax.experimental import pallas as pl
from jax.experimental.pallas import tpu as pltpu
from jax.experimental.pallas import tpu_sc as plsc

_N = 4096
_E = 131072
_NW = 32
_WORDS = _N * _N
_WPT = _WORDS // _NW
_ZB = 32768
_EPT = _E // _NW
_CH = 128
_NCHUNK = _EPT // _CH
_DPT = _N // _NW
_DCHUNK = 1


def _build_adj(src, dst):
    mesh = plsc.VectorSubcoreMesh(core_axis_name="c", subcore_axis_name="s")

    @functools.partial(
        pl.kernel, mesh=mesh,
        out_type=jax.ShapeDtypeStruct((_WORDS,), jnp.float32),
        scratch_types=[
            pltpu.VMEM((_ZB,), jnp.float32),
            pltpu.VMEM((_CH,), jnp.float32),
            pltpu.VMEM((_NCHUNK, _CH), jnp.int32),
            pltpu.VMEM((_DCHUNK, _CH), jnp.int32),
            pltpu.VMEM((_EPT,), jnp.int32),
            pltpu.VMEM((_EPT,), jnp.int32),
            pltpu.SemaphoreType.REGULAR,
            pltpu.SemaphoreType.DMA,
        ],
    )
    def adj_kernel(src_hbm, dst_hbm, out_hbm, zer_v, one_v, eidx_v, didx_v,
                   src_v, dst_v, xsem, sem):
        cid = lax.axis_index("c")
        tid = lax.axis_index("s")
        woff = cid * 16 + tid
        lanes = lax.iota(jnp.int32, 16)

        def fill_zeros(i, _):
            zer_v[pl.ds(i * 16, 16)] = jnp.zeros((16,), jnp.float32)
            return 0
        lax.fori_loop(0, _ZB // 16, fill_zeros, 0)

        zbase = woff * _WPT

        def zero_start(j, _):
            pltpu.async_copy(zer_v, out_hbm.at[pl.ds(zbase + j * _ZB, _ZB)],
                             sem)
            return 0
        lax.fori_loop(0, _WPT // _ZB, zero_start, 0)

        def fill_ones(i, _):
            one_v[pl.ds(i * 16, 16)] = jnp.ones((16,), jnp.float32)
            return 0
        lax.fori_loop(0, _CH // 16, fill_ones, 0)

        pltpu.sync_copy(src_hbm.at[pl.ds(woff * _EPT, _EPT)], src_v)
        pltpu.sync_copy(dst_hbm.at[pl.ds(woff * _EPT, _EPT)], dst_v)

        def fill_eidx(i, _):
            s16 = src_v[pl.ds(i * 16, 16)]
            d16 = dst_v[pl.ds(i * 16, 16)]
            r = i // (_CH // 16)
            c = (i % (_CH // 16)) * 16
            eidx_v[r, pl.ds(c, 16)] = s16 * _N + d16
            return 0
        lax.fori_loop(0, _EPT // 16, fill_eidx, 0)

        dbase = woff * _DPT

        def fill_didx(i, _):
            didx_v[0, pl.ds(i * 16, 16)] = (dbase + i * 16 + lanes) * (_N + 1)
            return 0
        lax.fori_loop(0, _DPT // 16, fill_didx, 0)

        def zero_drain(j, _):
            pltpu.make_async_copy(
                zer_v, out_hbm.at[pl.ds(zbase + j * _ZB, _ZB)], sem).wait()
            return 0
        lax.fori_loop(0, _WPT // _ZB, zero_drain, 0)

        plsc.subcore_barrier()

        @pl.when(tid == 0)
        def _handshake():
            pltpu.semaphore_signal(xsem, 1, core_index=1 - cid)
            pltpu.semaphore_wait(xsem, 1)

        plsc.subcore_barrier()

        pltpu.async_copy(one_v, out_hbm.at[didx_v.at[0]], sem)

        def scat_edge(r, _):
            pltpu.async_copy(one_v, out_hbm.at[eidx_v.at[r]], sem)
            return 0
        lax.fori_loop(0, _NCHUNK, scat_edge, 0)

        def scat_drain(r, _):
            pltpu.make_async_copy(one_v, out_hbm.at[didx_v.at[0]],
                                  sem).wait()
            return 0
        lax.fori_loop(0, _NCHUNK + _DCHUNK, scat_drain, 0)

    return adj_kernel(src, dst)


def _mm_bin(a, b, out_dtype, bm=512, bn=2048):
    grid = (_N // bn, _N // bm)

    def body(a_ref, b_ref, o_ref):
        o_ref[...] = (jnp.dot(a_ref[...], b_ref[...],
                              preferred_element_type=jnp.float32)
                      > 0).astype(out_dtype)

    return pl.pallas_call(
        body,
        grid=grid,
        in_specs=[
            pl.BlockSpec((bm, _N), lambda j, i: (i, 0)),
            pl.BlockSpec((_N, bn), lambda j, i: (0, j)),
        ],
        out_specs=pl.BlockSpec((bm, bn), lambda j, i: (i, j)),
        out_shape=jax.ShapeDtypeStruct((_N, _N), out_dtype),
        compiler_params=pltpu.CompilerParams(
            dimension_semantics=("parallel", "parallel"),
            vmem_limit_bytes=67043328),
    )(a, b)


def kernel(edge_index, num_nodes):
    ei = edge_index.astype(jnp.int32)
    adj_flat = _build_adj(ei[0], ei[1])
    adj = adj_flat.reshape(_N, _N).astype(jnp.bfloat16)
    hop2 = _mm_bin(adj, adj, jnp.bfloat16, bm=1024, bn=2048)
    return _mm_bin(hop2, adj, jnp.float32)


# speedup vs baseline: 1.0094x; 1.0094x over previous
import functools

import jax
import jax.numpy as jnp
from jax import lax
from jax.experimental import pallas as pl
from jax.experimental.pallas import tpu as pltpu
from jax.experimental.pallas import tpu_sc as plsc

_N = 4096
_E = 131072
_NW = 32
_WORDS = _N * _N
_WPT = _WORDS // _NW
_ZB = 16384
_EPT = _E // _NW
_CH = 128
_NCHUNK = _EPT // _CH
_DPT = _N // _NW
_DCHUNK = 1


def _build_adj(src, dst):
    mesh = plsc.VectorSubcoreMesh(core_axis_name="c", subcore_axis_name="s")

    @functools.partial(
        pl.kernel, mesh=mesh,
        out_type=jax.ShapeDtypeStruct((_WORDS,), jnp.float32),
        scratch_types=[
            pltpu.VMEM((_ZB,), jnp.float32),
            pltpu.VMEM((_CH,), jnp.float32),
            pltpu.VMEM((_NCHUNK, _CH), jnp.int32),
            pltpu.VMEM((_DCHUNK, _CH), jnp.int32),
            pltpu.VMEM((_EPT,), jnp.int32),
            pltpu.VMEM((_EPT,), jnp.int32),
            pltpu.SemaphoreType.REGULAR,
            pltpu.SemaphoreType.DMA,
        ],
    )
    def adj_kernel(src_hbm, dst_hbm, out_hbm, zer_v, one_v, eidx_v, didx_v,
                   src_v, dst_v, xsem, sem):
        cid = lax.axis_index("c")
        tid = lax.axis_index("s")
        woff = cid * 16 + tid
        lanes = lax.iota(jnp.int32, 16)

        def fill_zeros(i, _):
            zer_v[pl.ds(i * 16, 16)] = jnp.zeros((16,), jnp.float32)
            return 0
        lax.fori_loop(0, _ZB // 16, fill_zeros, 0)

        zbase = woff * _WPT

        def zero_start(j, _):
            pltpu.async_copy(zer_v, out_hbm.at[pl.ds(zbase + j * _ZB, _ZB)],
                             sem)
            return 0
        lax.fori_loop(0, _WPT // _ZB, zero_start, 0)

        def fill_ones(i, _):
            one_v[pl.ds(i * 16, 16)] = jnp.ones((16,), jnp.float32)
            return 0
        lax.fori_loop(0, _CH // 16, fill_ones, 0)

        pltpu.sync_copy(src_hbm.at[pl.ds(woff * _EPT, _EPT)], src_v)
        pltpu.sync_copy(dst_hbm.at[pl.ds(woff * _EPT, _EPT)], dst_v)

        def fill_eidx(i, _):
            s16 = src_v[pl.ds(i * 16, 16)]
            d16 = dst_v[pl.ds(i * 16, 16)]
            r = i // (_CH // 16)
            c = (i % (_CH // 16)) * 16
            eidx_v[r, pl.ds(c, 16)] = s16 * _N + d16
            return 0
        lax.fori_loop(0, _EPT // 16, fill_eidx, 0)

        dbase = woff * _DPT

        def fill_didx(i, _):
            didx_v[0, pl.ds(i * 16, 16)] = (dbase + i * 16 + lanes) * (_N + 1)
            return 0
        lax.fori_loop(0, _DPT // 16, fill_didx, 0)

        def zero_drain(j, _):
            pltpu.make_async_copy(
                zer_v, out_hbm.at[pl.ds(zbase + j * _ZB, _ZB)], sem).wait()
            return 0
        lax.fori_loop(0, _WPT // _ZB, zero_drain, 0)

        plsc.subcore_barrier()

        @pl.when(tid == 0)
        def _handshake():
            pltpu.semaphore_signal(xsem, 1, core_index=1 - cid)
            pltpu.semaphore_wait(xsem, 1)

        plsc.subcore_barrier()

        pltpu.async_copy(one_v, out_hbm.at[didx_v.at[0]], sem)

        def scat_edge(r, _):
            pltpu.async_copy(one_v, out_hbm.at[eidx_v.at[r]], sem)
            return 0
        lax.fori_loop(0, _NCHUNK, scat_edge, 0)

        def scat_drain(r, _):
            pltpu.make_async_copy(one_v, out_hbm.at[didx_v.at[0]],
                                  sem).wait()
            return 0
        lax.fori_loop(0, _NCHUNK + _DCHUNK, scat_drain, 0)

    return adj_kernel(src, dst)


def _mm_bin(a, b, out_dtype, bm=512, bn=2048):
    grid = (_N // bn, _N // bm)

    def body(a_ref, b_ref, o_ref):
        o_ref[...] = (jnp.dot(a_ref[...], b_ref[...],
                              preferred_element_type=jnp.float32)
                      > 0).astype(out_dtype)

    return pl.pallas_call(
        body,
        grid=grid,
        in_specs=[
            pl.BlockSpec((bm, _N), lambda j, i: (i, 0)),
            pl.BlockSpec((_N, bn), lambda j, i: (0, j)),
        ],
        out_specs=pl.BlockSpec((bm, bn), lambda j, i: (i, j)),
        out_shape=jax.ShapeDtypeStruct((_N, _N), out_dtype),
        compiler_params=pltpu.CompilerParams(
            dimension_semantics=("parallel", "parallel"),
            vmem_limit_bytes=67043328),
    )(a, b)


def kernel(edge_index, num_nodes):
    ei = edge_index.astype(jnp.int32)
    adj_flat = _build_adj(ei[0], ei[1])
    adj = adj_flat.reshape(_N, _N).astype(jnp.bfloat16)
    hop2 = _mm_bin(adj, adj, jnp.bfloat16, bm=1024, bn=2048)
    return _mm_bin(hop2, adj, jnp.float32)


# speedup vs baseline: 1.0104x; 1.0009x over previous
import functools

import jax
import jax.numpy as jnp
from jax import lax
from jax.experimental import pallas as pl
from jax.experimental.pallas import tpu as pltpu
from jax.experimental.pallas import tpu_sc as plsc

_N = 4096
_E = 131072
_NW = 32
_WORDS = _N * _N
_WPT = _WORDS // _NW
_ZB = 16384
_EPT = _E // _NW
_CH = 128
_NCHUNK = _EPT // _CH
_DPT = _N // _NW
_DCHUNK = 1


def _build_adj(src, dst):
    mesh = plsc.VectorSubcoreMesh(core_axis_name="c", subcore_axis_name="s")

    @functools.partial(
        pl.kernel, mesh=mesh,
        out_type=jax.ShapeDtypeStruct((_WORDS,), jnp.float32),
        scratch_types=[
            pltpu.VMEM((_ZB,), jnp.float32),
            pltpu.VMEM((_CH,), jnp.float32),
            pltpu.VMEM((_NCHUNK, _CH), jnp.int32),
            pltpu.VMEM((_DCHUNK, _CH), jnp.int32),
            pltpu.VMEM((_EPT,), jnp.int32),
            pltpu.VMEM((_EPT,), jnp.int32),
            pltpu.SemaphoreType.REGULAR,
            pltpu.SemaphoreType.DMA,
        ],
    )
    def adj_kernel(src_hbm, dst_hbm, out_hbm, zer_v, one_v, eidx_v, didx_v,
                   src_v, dst_v, xsem, sem):
        cid = lax.axis_index("c")
        tid = lax.axis_index("s")
        woff = cid * 16 + tid
        lanes = lax.iota(jnp.int32, 16)

        def fill_zeros(i, _):
            zer_v[pl.ds(i * 16, 16)] = jnp.zeros((16,), jnp.float32)
            return 0
        lax.fori_loop(0, _ZB // 16, fill_zeros, 0)

        zbase = woff * _WPT

        def zero_start(j, _):
            pltpu.async_copy(zer_v, out_hbm.at[pl.ds(zbase + j * _ZB, _ZB)],
                             sem)
            return 0
        lax.fori_loop(0, _WPT // _ZB, zero_start, 0)

        def fill_ones(i, _):
            one_v[pl.ds(i * 16, 16)] = jnp.ones((16,), jnp.float32)
            return 0
        lax.fori_loop(0, _CH // 16, fill_ones, 0)

        pltpu.sync_copy(src_hbm.at[pl.ds(woff * _EPT, _EPT)], src_v)
        pltpu.sync_copy(dst_hbm.at[pl.ds(woff * _EPT, _EPT)], dst_v)

        def fill_eidx(i, _):
            s16 = src_v[pl.ds(i * 16, 16)]
            d16 = dst_v[pl.ds(i * 16, 16)]
            r = i // (_CH // 16)
            c = (i % (_CH // 16)) * 16
            eidx_v[r, pl.ds(c, 16)] = s16 * _N + d16
            return 0
        lax.fori_loop(0, _EPT // 16, fill_eidx, 0)

        dbase = woff * _DPT

        def fill_didx(i, _):
            didx_v[0, pl.ds(i * 16, 16)] = (dbase + i * 16 + lanes) * (_N + 1)
            return 0
        lax.fori_loop(0, _DPT // 16, fill_didx, 0)

        def zero_drain(j, _):
            pltpu.make_async_copy(
                zer_v, out_hbm.at[pl.ds(zbase + j * _ZB, _ZB)], sem).wait()
            return 0
        lax.fori_loop(0, _WPT // _ZB, zero_drain, 0)

        plsc.subcore_barrier()

        @pl.when(tid == 0)
        def _handshake():
            pltpu.semaphore_signal(xsem, 1, core_index=1 - cid)
            pltpu.semaphore_wait(xsem, 1)

        plsc.subcore_barrier()

        pltpu.async_copy(one_v, out_hbm.at[didx_v.at[0]], sem)

        def scat_edge(r, _):
            pltpu.async_copy(one_v, out_hbm.at[eidx_v.at[r]], sem)
            return 0
        lax.fori_loop(0, _NCHUNK, scat_edge, 0)

        def scat_drain(r, _):
            pltpu.make_async_copy(one_v, out_hbm.at[didx_v.at[0]],
                                  sem).wait()
            return 0
        lax.fori_loop(0, _NCHUNK + _DCHUNK, scat_drain, 0)

    return adj_kernel(src, dst)


def _mm_bin(a, b, out_dtype, bm=512, bn=2048):
    grid = (_N // bn, _N // bm)

    def body(a_ref, b_ref, o_ref):
        o_ref[...] = (jnp.dot(a_ref[...], b_ref[...],
                              preferred_element_type=jnp.float32)
                      > 0).astype(out_dtype)

    return pl.pallas_call(
        body,
        grid=grid,
        in_specs=[
            pl.BlockSpec((bm, _N), lambda j, i: (i, 0)),
            pl.BlockSpec((_N, bn), lambda j, i: (0, j)),
        ],
        out_specs=pl.BlockSpec((bm, bn), lambda j, i: (i, j)),
        out_shape=jax.ShapeDtypeStruct((_N, _N), out_dtype),
        compiler_params=pltpu.CompilerParams(
            dimension_semantics=("parallel", "parallel")),
    )(a, b)


def kernel(edge_index, num_nodes):
    ei = edge_index.astype(jnp.int32)
    adj_flat = _build_adj(ei[0], ei[1])
    adj = adj_flat.reshape(_N, _N).astype(jnp.bfloat16)
    hop2 = _mm_bin(adj, adj, jnp.bfloat16)
    return _mm_bin(hop2, adj, jnp.float32)
